# Initial kernel scaffold; baseline (speedup 1.0000x reference)
import jax
import jax.numpy as jnp
from jax.experimental import pallas as pl


def kernel(x, edge_index, edge_weight, weight, bias):
    raise NotImplementedError("write your pallas kernel here")


# speedup vs baseline: 2.9977x; 2.9977x over previous
import functools

import jax
import jax.numpy as jnp
from jax import lax
from jax.experimental import pallas as pl
from jax.experimental.pallas import tpu as pltpu
from jax.experimental.pallas import tpu_sc as plsc

_N = 10000
_E = 320000
_D = 128
_HALF = _D // 2
_NSUB = 16
_EPT = _E // _NSUB
_NB = 80
_NBLK = _EPT // _NB
_ROWS_T = 624
_TAIL0 = _NSUB * _ROWS_T
_TAIL = _N - _TAIL0
_INIT_ROWS = 104
_INIT_CHUNKS = _ROWS_T // _INIT_ROWS
_LANES = 16


def _matmul_body(x_ref, w_ref, out_ref):
    out_ref[...] = jnp.dot(
        x_ref[...], w_ref[0],
        preferred_element_type=jnp.float32,
        precision=lax.Precision.HIGHEST,
    )


def _support_halves(x, weight_halves):
    return pl.pallas_call(
        _matmul_body,
        grid=(2,),
        in_specs=[
            pl.BlockSpec((_N, _D), lambda c: (0, 0)),
            pl.BlockSpec((1, _D, _HALF), lambda c: (c, 0, 0)),
        ],
        out_specs=pl.BlockSpec((_N, _HALF), lambda c: (c, 0)),
        out_shape=jax.ShapeDtypeStruct((2 * _N, _HALF), jnp.float32),
    )(x, weight_halves)


def _sc_body(support, src, dst, ew, bias, out,
             srcv, dstv, wv, rows, bbuf, bvec, acc):
    c = lax.axis_index("c")
    s = lax.axis_index("s")

    pltpu.sync_copy(src.at[c * _NSUB + s], srcv)
    pltpu.sync_copy(dst.at[s], dstv)
    pltpu.sync_copy(ew.at[s], wv)

    pltpu.sync_copy(bias.at[pl.ds(c * _HALF, _HALF)], bvec)
    bregs = [bvec[pl.ds(j * _LANES, _LANES)] for j in range(_HALF // _LANES)]

    def fill_row(r, carry):
        for j in range(_HALF // _LANES):
            bbuf[r, pl.ds(j * _LANES, _LANES)] = bregs[j]
        return carry

    lax.fori_loop(0, _INIT_ROWS, fill_row, 0)
    for k in range(_INIT_CHUNKS):
        pltpu.sync_copy(
            bbuf, acc.at[pl.ds(s * _ROWS_T + k * _INIT_ROWS, _INIT_ROWS)])

    @pl.when(s == _NSUB - 1)
    def _init_tail():
        pltpu.sync_copy(bbuf.at[pl.ds(0, _TAIL)],
                        acc.at[pl.ds(_TAIL0, _TAIL)])

    plsc.subcore_barrier()

    def blk(b, carry):
        pltpu.sync_copy(support.at[srcv.at[b]], rows)

        def group(g, gcarry):
            wv16 = wv[pl.ds(b * _NB + g * _LANES, _LANES)]
            for e in range(_LANES):
                wsplat = jnp.broadcast_to(wv16[e], (_LANES,))
                base = g * _LANES + e
                for j in range(_HALF // _LANES):
                    sl = pl.ds(j * _LANES, _LANES)
                    rows[base, sl] = rows[base, sl] * wsplat
            return gcarry

        lax.fori_loop(0, _NB // _LANES, group, 0)
        pltpu.sync_copy(rows, acc.at[dstv.at[b]], add=True)
        return carry

    lax.fori_loop(0, _NBLK, blk, 0)
    plsc.subcore_barrier()

    pltpu.sync_copy(acc.at[pl.ds(s * _ROWS_T, _ROWS_T)],
                    out.at[c, pl.ds(s * _ROWS_T, _ROWS_T)])

    @pl.when(s == _NSUB - 1)
    def _write_tail():
        pltpu.sync_copy(acc.at[pl.ds(_TAIL0, _TAIL)],
                        out.at[c, pl.ds(_TAIL0, _TAIL)])


_sc_call = functools.partial(
    pl.kernel,
    out_type=jax.ShapeDtypeStruct((2, _N, _HALF), jnp.float32),
    mesh=plsc.VectorSubcoreMesh(
        core_axis_name="c", subcore_axis_name="s",
        num_cores=2, num_subcores=16),
    scratch_types=[
        pltpu.VMEM((_NBLK, _NB), jnp.int32),
        pltpu.VMEM((_NBLK, _NB), jnp.int32),
        pltpu.VMEM((_EPT,), jnp.float32),
        pltpu.VMEM((_NB, _HALF), jnp.float32),
        pltpu.VMEM((_INIT_ROWS, _HALF), jnp.float32),
        pltpu.VMEM((_HALF,), jnp.float32),
        pltpu.VMEM_SHARED((_N, _HALF), jnp.float32),
    ],
    compiler_params=pltpu.CompilerParams(use_tc_tiling_on_sc=False),
)(_sc_body)


@jax.jit
def kernel(x, edge_index, edge_weight, weight, bias):
    weight_halves = weight.reshape(_D, 2, _HALF).transpose(1, 0, 2)
    support2 = _support_halves(x, weight_halves)
    src = edge_index[1].reshape(_NSUB, _NBLK, _NB)
    src_all = jnp.concatenate([src, src + _N], axis=0)
    dst = edge_index[0].reshape(_NSUB, _NBLK, _NB)
    ew = edge_weight.reshape(_NSUB, _EPT)
    out2 = _sc_call(support2, src_all, dst, ew, bias)
    return out2.transpose(1, 0, 2).reshape(_N, _D)
